# baseline (speedup 1.0000x reference)
import jax
import jax.numpy as jnp
from jax import lax
from jax.experimental import pallas as pl
from jax.experimental.pallas import tpu as pltpu
from jax.experimental.pallas import tpu_sc as plsc

B, D, H = 8192, 512, 512
BLK = 512
NBLK = B // BLK

NC, NS, L = 2, 16, 16
NW = NC * NS
SEG = B // NW
SUB = 64
PPAD = B + 2 * L


def _wid():
    return lax.axis_index("s") * NC + lax.axis_index("c")


def _prep_body(m_ref, tgt_ref, cnt_ref):
    m = m_ref[...]
    cv = m
    sh = 1
    while sh < B:
        z = jnp.zeros((1, sh), cv.dtype)
        cv = cv + jnp.concatenate([z, cv[:, :B - sh]], axis=1)
        sh *= 2
    nv = cv[0, B - 1]
    pos = cv - m
    i = jax.lax.broadcasted_iota(jnp.int32, (1, B), 1)
    tgt_ref[...] = jnp.where(m == 1, pos, nv + (i - pos))
    cnt_ref[...] = jnp.full((1, L), nv, jnp.int32)


def _tc_prep(mask):
    tgt, cnt = pl.pallas_call(
        _prep_body,
        out_shape=[jax.ShapeDtypeStruct((1, B), jnp.int32),
                   jax.ShapeDtypeStruct((1, L), jnp.int32)],
    )(mask.reshape(1, B))
    return tgt.reshape(B), cnt.reshape(L)


def _sc_perm(tgt_hbm, perm_hbm, tgt_v, idx_v, sem):
    wid = _wid()
    base = wid * SEG
    pltpu.sync_copy(tgt_hbm.at[pl.ds(base, SEG)], tgt_v)
    iota = lax.iota(jnp.int32, L)
    for k in range(SEG // L):
        idx_v[pl.ds(k * L, L)] = iota + (base + k * L)
    for k in range(SEG // L):
        t = tgt_v[pl.ds(k * L, L)]
        pltpu.async_copy(idx_v.at[pl.ds(k * L, L)], perm_hbm.at[t], sem).wait()


def _sc_gather(x_hbm, h_hbm, c_hbm, perm_hbm, cnt_hbm,
               xg_hbm, hg_hbm, cg_hbm, idx_v, c16, rows, sem):
    wid = _wid()
    base = wid * SEG
    pltpu.sync_copy(perm_hbm.at[pl.ds(base, SEG)], idx_v)
    pltpu.sync_copy(cnt_hbm, c16)
    nv = c16[pl.ds(0, L)][0]
    for (src, dst) in ((x_hbm, xg_hbm), (h_hbm, hg_hbm), (c_hbm, cg_hbm)):
        for s in range(SEG // SUB):
            sb = base + s * SUB

            @pl.when(sb < nv)
            def _(src=src, dst=dst, s=s, sb=sb):
                pltpu.async_copy(src.at[idx_v.at[pl.ds(s * SUB, SUB)]],
                                 rows, sem).wait()
                pltpu.sync_copy(rows, dst.at[pl.ds(sb, SUB)])


def _lstm_block(cnt_ref, x_ref, h_ref, c_ref, wih_ref, whh_ref,
                bih_ref, bhh_ref, ho_ref, co_ref):
    nvb = (cnt_ref[0] + BLK - 1) // BLK

    @pl.when(pl.program_id(0) < nvb)
    def _():
        dn = (((1,), (1,)), ((), ()))
        gates = lax.dot_general(x_ref[...], wih_ref[...], dn,
                                preferred_element_type=jnp.float32)
        gates = gates + lax.dot_general(h_ref[...], whh_ref[...], dn,
                                        preferred_element_type=jnp.float32)
        gates = gates + (bih_ref[...] + bhh_ref[...])
        i = jax.nn.sigmoid(gates[:, 0 * H:1 * H])
        f = jax.nn.sigmoid(gates[:, 1 * H:2 * H])
        g = jnp.tanh(gates[:, 2 * H:3 * H])
        o = jax.nn.sigmoid(gates[:, 3 * H:4 * H])
        c_new = f * c_ref[...] + i * g
        ho_ref[...] = o * jnp.tanh(c_new)
        co_ref[...] = c_new


def _tc_lstm(cnt, xg, hg, cg, W_ih, W_hh, b_ih, b_hh):
    def blk(i, cnt_ref):
        nvb = (cnt_ref[0] + BLK - 1) // BLK
        return (jnp.clip(i, 0, jnp.maximum(nvb - 1, 0)), 0)

    def whole(i, cnt_ref):
        return (0, 0)

    return pl.pallas_call(
        _lstm_block,
        grid_spec=pltpu.PrefetchScalarGridSpec(
            num_scalar_prefetch=1,
            grid=(NBLK,),
            in_specs=[
                pl.BlockSpec((BLK, D), blk),
                pl.BlockSpec((BLK, H), blk),
                pl.BlockSpec((BLK, H), blk),
                pl.BlockSpec((4 * H, D), whole),
                pl.BlockSpec((4 * H, H), whole),
                pl.BlockSpec((1, 4 * H), whole),
                pl.BlockSpec((1, 4 * H), whole),
            ],
            out_specs=[
                pl.BlockSpec((BLK, H), blk),
                pl.BlockSpec((BLK, H), blk),
            ],
        ),
        out_shape=[
            jax.ShapeDtypeStruct((B, H), jnp.float32),
            jax.ShapeDtypeStruct((B, H), jnp.float32),
        ],
    )(cnt, xg, hg, cg, W_ih, W_hh, b_ih.reshape(1, 4 * H),
      b_hh.reshape(1, 4 * H))


def _sc_scatter(hn_hbm, cn_hbm, h0_hbm, c0_hbm, perm_hbm, cnt_hbm,
                oh_hbm, oc_hbm, idx2, c16, bufa, bufb, sem):
    wid = _wid()
    base = wid * SEG
    for s in range(SEG // SUB):
        pltpu.sync_copy(perm_hbm.at[pl.ds(base + s * SUB, SUB)], idx2.at[s])
    pltpu.sync_copy(cnt_hbm, c16)
    nv = c16[pl.ds(0, L)][0]

    for (comp, old, out) in ((hn_hbm, h0_hbm, oh_hbm), (cn_hbm, c0_hbm, oc_hbm)):
        for s in range(SEG // SUB):
            sb = base + s * SUB

            @pl.when(sb + SUB <= nv)
            def _(comp=comp, out=out, s=s, sb=sb):
                pltpu.sync_copy(comp.at[pl.ds(sb, SUB)], bufa)
                pltpu.async_copy(bufa, out.at[idx2.at[s]], sem).wait()

            @pl.when(sb >= nv)
            def _(old=old, out=out, s=s):
                pltpu.async_copy(old.at[idx2.at[s]], bufa, sem).wait()
                pltpu.async_copy(bufa, out.at[idx2.at[s]], sem).wait()

            @pl.when(jnp.logical_and(sb < nv, sb + SUB > nv))
            def _(comp=comp, old=old, out=out, s=s, sb=sb):
                pltpu.async_copy(old.at[idx2.at[s]], bufa, sem).wait()
                pltpu.sync_copy(comp.at[pl.ds(sb, SUB)], bufb)

                def rowfix(j, _):
                    @pl.when(sb + j < nv)
                    def _():
                        for t in range(H // L):
                            bufa[j, pl.ds(t * L, L)] = bufb[j, pl.ds(t * L, L)]
                    return 0

                lax.fori_loop(0, SUB, rowfix, 0)
                pltpu.async_copy(bufa, out.at[idx2.at[s]], sem).wait()


def kernel(x, mask, h0, c0, W_ih, W_hh, b_ih, b_hh):
    tgt, cnt = _tc_prep(mask)
    mesh = plsc.VectorSubcoreMesh(core_axis_name="c", subcore_axis_name="s",
                                  num_cores=NC, num_subcores=NS)

    perm = pl.kernel(
        _sc_perm, mesh=mesh,
        out_type=jax.ShapeDtypeStruct((PPAD,), jnp.int32),
        scratch_types=[pltpu.VMEM((SEG,), jnp.int32),
                       pltpu.VMEM((SEG,), jnp.int32),
                       pltpu.SemaphoreType.DMA],
    )(tgt)

    xg, hg, cg = pl.kernel(
        _sc_gather, mesh=mesh,
        out_type=[jax.ShapeDtypeStruct((B, D), jnp.float32),
                  jax.ShapeDtypeStruct((B, H), jnp.float32),
                  jax.ShapeDtypeStruct((B, H), jnp.float32)],
        scratch_types=[pltpu.VMEM((SEG,), jnp.int32),
                       pltpu.VMEM((L,), jnp.int32),
                       pltpu.VMEM((SUB, D), jnp.float32),
                       pltpu.SemaphoreType.DMA],
    )(x, h0, c0, perm, cnt)

    hn, cn = _tc_lstm(cnt, xg, hg, cg, W_ih, W_hh, b_ih, b_hh)

    oh, oc = pl.kernel(
        _sc_scatter, mesh=mesh,
        out_type=[jax.ShapeDtypeStruct((B, H), jnp.float32),
                  jax.ShapeDtypeStruct((B, H), jnp.float32)],
        scratch_types=[pltpu.VMEM((SEG // SUB, SUB), jnp.int32),
                       pltpu.VMEM((L,), jnp.int32),
                       pltpu.VMEM((SUB, H), jnp.float32),
                       pltpu.VMEM((SUB, H), jnp.float32),
                       pltpu.SemaphoreType.DMA],
    )(hn, cn, h0, c0, perm, cnt)

    return oh, oc


# speedup vs baseline: 3.2060x; 3.2060x over previous
import jax
import jax.numpy as jnp
from jax.experimental import pallas as pl
from jax.experimental.pallas import tpu as pltpu

B, D, H = 8192, 512, 512
BLK = 1024


def _lstm_block(x_ref, h_ref, c_ref, m_ref, wih_ref, whh_ref,
                bih_ref, bhh_ref, ho_ref, co_ref):
    dn = (((1,), (1,)), ((), ()))
    gates = jax.lax.dot_general(x_ref[...], wih_ref[...], dn,
                                preferred_element_type=jnp.float32)
    gates = gates + jax.lax.dot_general(h_ref[...], whh_ref[...], dn,
                                        preferred_element_type=jnp.float32)
    gates = gates + (bih_ref[...] + bhh_ref[...])
    i = jax.nn.sigmoid(gates[:, 0 * H:1 * H])
    f = jax.nn.sigmoid(gates[:, 1 * H:2 * H])
    g = jnp.tanh(gates[:, 2 * H:3 * H])
    o = jax.nn.sigmoid(gates[:, 3 * H:4 * H])
    c_old = c_ref[...]
    c_new = f * c_old + i * g
    h_new = o * jnp.tanh(c_new)
    valid = m_ref[...] == 1
    ho_ref[...] = jnp.where(valid, h_new, h_ref[...])
    co_ref[...] = jnp.where(valid, c_new, c_old)


def kernel(x, mask, h0, c0, W_ih, W_hh, b_ih, b_hh):
    mask2 = mask.reshape(B, 1)
    bih2 = b_ih.reshape(1, 4 * H)
    bhh2 = b_hh.reshape(1, 4 * H)
    ho, co = pl.pallas_call(
        _lstm_block,
        grid=(B // BLK,),
        in_specs=[
            pl.BlockSpec((BLK, D), lambda i: (i, 0)),
            pl.BlockSpec((BLK, H), lambda i: (i, 0)),
            pl.BlockSpec((BLK, H), lambda i: (i, 0)),
            pl.BlockSpec((BLK, 1), lambda i: (i, 0)),
            pl.BlockSpec((4 * H, D), lambda i: (0, 0)),
            pl.BlockSpec((4 * H, H), lambda i: (0, 0)),
            pl.BlockSpec((1, 4 * H), lambda i: (0, 0)),
            pl.BlockSpec((1, 4 * H), lambda i: (0, 0)),
        ],
        out_specs=[
            pl.BlockSpec((BLK, H), lambda i: (i, 0)),
            pl.BlockSpec((BLK, H), lambda i: (i, 0)),
        ],
        out_shape=[
            jax.ShapeDtypeStruct((B, H), jnp.float32),
            jax.ShapeDtypeStruct((B, H), jnp.float32),
        ],
        compiler_params=pltpu.CompilerParams(
            dimension_semantics=("parallel",),
        ),
    )(x, h0, c0, mask2, W_ih, W_hh, bih2, bhh2)
    return ho, co
